# baseline (speedup 1.0000x reference)
import jax
import jax.numpy as jnp
from jax import lax
from jax.experimental import pallas as pl
from jax.experimental.pallas import tpu as pltpu
from jax.experimental.pallas import tpu_sc as plsc

BATCH = 16384
LATENT = 128
NC = 2
NS = 16
L = 16
NW = NC * NS
BPW = BATCH // NW
CHUNK = 128
NCHUNK = BPW // CHUNK
GROUPS = CHUNK // L
UNROLL = 16

_MESH = dict(core_axis_name="c", subcore_axis_name="s")


def _dot_body(uid_hbm, iid_hbm, avg_hbm, p_hbm, q_hbm,
              out_hbm, ids, rows, small, sem0, sem1, sem_ids, sem_out):
    wid = lax.axis_index("s") * NC + lax.axis_index("c")
    base = wid * BPW
    sem_slot = (sem0, sem1)
    AVG, OUT = 0, 1

    def chunk_copies(k, buf):
        ck = pl.ds(k * CHUNK, CHUNK)
        sem = sem_slot[buf]
        return (pltpu.make_async_copy(p_hbm.at[ids.at[0, ck]],
                                      rows.at[2 * buf], sem),
                pltpu.make_async_copy(q_hbm.at[ids.at[1, ck]],
                                      rows.at[2 * buf + 1], sem))

    def issue(k, buf):
        for cp in chunk_copies(k, buf):
            cp.start()

    def drain(k, buf):
        for cp in chunk_copies(k, buf):
            cp.wait()

    cp_u0 = pltpu.async_copy(uid_hbm.at[pl.ds(base, CHUNK)],
                             ids.at[0, pl.ds(0, CHUNK)], sem_ids)
    cp_i0 = pltpu.async_copy(iid_hbm.at[pl.ds(base, CHUNK)],
                             ids.at[1, pl.ds(0, CHUNK)], sem_ids)
    cp_u0.wait()
    cp_i0.wait()
    issue(0, 0)

    rest = BPW - CHUNK
    cp_ur = pltpu.async_copy(uid_hbm.at[pl.ds(base + CHUNK, rest)],
                             ids.at[0, pl.ds(CHUNK, rest)], sem_ids)
    cp_ir = pltpu.async_copy(iid_hbm.at[pl.ds(base + CHUNK, rest)],
                             ids.at[1, pl.ds(CHUNK, rest)], sem_ids)
    cp_av = pltpu.async_copy(avg_hbm.at[pl.ds(base, BPW)],
                             small.at[AVG], sem_ids)
    cp_ur.wait()
    cp_ir.wait()
    cp_av.wait()
    issue(1, 1)

    lane = lax.iota(jnp.int32, L)
    av16 = jnp.full((L,), AVG, jnp.int32)
    ot16 = jnp.full((L,), OUT, jnp.int32)

    def compute_chunk(k, buf):
        p_rows, q_rows = rows.at[2 * buf], rows.at[2 * buf + 1]

        @pl.loop(0, GROUPS)
        def _(g):
            rows16 = lane + g * L

            def dbody(m, accs, rows16=rows16):
                a0, a1 = accs
                c0 = m * UNROLL
                for u in range(UNROLL):
                    col = (lane + (c0 + u)) & (LATENT - 1)
                    pv = plsc.load_gather(p_rows, [rows16, col])
                    qv = plsc.load_gather(q_rows, [rows16, col])
                    if u % 2 == 0:
                        a0 = a0 + pv * qv
                    else:
                        a1 = a1 + pv * qv
                return a0, a1

            zero = jnp.zeros((L,), jnp.float32)
            a0, a1 = lax.fori_loop(0, LATENT // UNROLL, dbody, (zero, zero))
            bidx = rows16 + k * CHUNK
            avv = plsc.load_gather(small, [av16, bidx])
            plsc.store_scatter(small, [ot16, bidx], (a0 + a1) + avv)

        pltpu.make_async_copy(
            small.at[OUT, pl.ds(k * CHUNK, CHUNK)],
            out_hbm.at[pl.ds(base + k * CHUNK, CHUNK)], sem_out).start()

    @pl.loop(0, NCHUNK, step=2)
    def _(kb):
        for half in range(2):
            k = kb + half
            drain(k, half)
            compute_chunk(k, half)

            @pl.when(k + 2 < NCHUNK)
            def _():
                issue(k + 2, half)

    for k in range(NCHUNK):
        pltpu.make_async_copy(
            small.at[OUT, pl.ds(k * CHUNK, CHUNK)],
            out_hbm.at[pl.ds(base + k * CHUNK, CHUNK)], sem_out).wait()


def _bias_body(uid_hbm, iid_hbm, ub_hbm, ib_hbm, ain_hbm,
               out_hbm, ids, small, sem_ids, sem_b, sem_out):
    wid = lax.axis_index("s") * NC + lax.axis_index("c")
    base = wid * BPW
    UB, IB, AIN, OUT = 0, 1, 2, 3

    cp_u = pltpu.async_copy(uid_hbm.at[pl.ds(base, BPW)], ids.at[0], sem_ids)
    cp_i = pltpu.async_copy(iid_hbm.at[pl.ds(base, BPW)], ids.at[1], sem_ids)
    cp_a = pltpu.async_copy(ain_hbm.at[pl.ds(base, BPW)], small.at[AIN],
                            sem_ids)
    cp_u.wait()
    cp_i.wait()
    bias_cps = []
    for k in range(NCHUNK):
        ck = pl.ds(k * CHUNK, CHUNK)
        bias_cps.append(pltpu.make_async_copy(
            ub_hbm.at[ids.at[0, ck]], small.at[UB, ck], sem_b))
        bias_cps.append(pltpu.make_async_copy(
            ib_hbm.at[ids.at[1, ck]], small.at[IB, ck], sem_b))
    for cp in bias_cps:
        cp.start()
    cp_a.wait()
    for cp in bias_cps:
        cp.wait()

    lane = lax.iota(jnp.int32, L)
    ub16 = jnp.full((L,), UB, jnp.int32)
    ib16 = jnp.full((L,), IB, jnp.int32)
    ai16 = jnp.full((L,), AIN, jnp.int32)
    ot16 = jnp.full((L,), OUT, jnp.int32)

    @pl.loop(0, BPW // L)
    def _(g):
        idx = lane + g * L
        tot = (plsc.load_gather(small, [ai16, idx])
               + plsc.load_gather(small, [ub16, idx])
               + plsc.load_gather(small, [ib16, idx]))
        plsc.store_scatter(small, [ot16, idx], tot)

    cp_o = pltpu.make_async_copy(small.at[OUT], out_hbm.at[pl.ds(base, BPW)],
                                 sem_out)
    cp_o.start()
    cp_o.wait()


def _mf(user_id, item_id, avg, P, Q, ub, ib):
    mesh = plsc.VectorSubcoreMesh(**_MESH)
    dot = pl.kernel(
        _dot_body,
        mesh=mesh,
        compiler_params=pltpu.CompilerParams(needs_layout_passes=False),
        out_type=jax.ShapeDtypeStruct((BATCH,), jnp.float32),
        scratch_types=[
            pltpu.VMEM((2, BPW), jnp.int32),
            pltpu.VMEM((4, CHUNK, LATENT), jnp.float32),
            pltpu.VMEM((2, BPW), jnp.float32),
            pltpu.SemaphoreType.DMA,
            pltpu.SemaphoreType.DMA,
            pltpu.SemaphoreType.DMA,
            pltpu.SemaphoreType.DMA,
        ],
    )(user_id, item_id, avg, P, Q)
    mesh2 = plsc.VectorSubcoreMesh(**_MESH)
    return pl.kernel(
        _bias_body,
        mesh=mesh2,
        compiler_params=pltpu.CompilerParams(needs_layout_passes=False),
        out_type=jax.ShapeDtypeStruct((BATCH,), jnp.float32),
        scratch_types=[
            pltpu.VMEM((2, BPW), jnp.int32),
            pltpu.VMEM((4, BPW), jnp.float32),
            pltpu.SemaphoreType.DMA,
            pltpu.SemaphoreType.DMA,
            pltpu.SemaphoreType.DMA,
        ],
    )(user_id, item_id, ub, ib, dot)


def kernel(user_id, item_id, avg_score, P, Q, user_bias, item_bias):
    out = _mf(user_id.astype(jnp.int32), item_id.astype(jnp.int32),
              avg_score.reshape(-1), P, Q,
              user_bias.reshape(-1), item_bias.reshape(-1))
    return out.reshape(BATCH, 1)


# speedup vs baseline: 1.1059x; 1.1059x over previous
import jax
import jax.numpy as jnp
from jax import lax
from jax.experimental import pallas as pl
from jax.experimental.pallas import tpu as pltpu
from jax.experimental.pallas import tpu_sc as plsc

BATCH = 16384
LATENT = 128
NC = 2
NS = 16
L = 16
NW = NC * NS
BPW = BATCH // NW
CHUNK = 128
NCHUNK = BPW // CHUNK
GROUPS = CHUNK // L
UNROLL = 16


def _mf_body(uid_hbm, iid_hbm, avg_hbm, p_hbm, q_hbm, ub_hbm, ib_hbm,
             out_hbm, ids, rows, small, sem0, sem1, sem_ids, sem_out):
    wid = lax.axis_index("s") * NC + lax.axis_index("c")
    base = wid * BPW
    sem_slot = (sem0, sem1)
    UB, IB, AVG, OUT = 0, 1, 2, 3

    def chunk_copies(k, buf):
        ck = pl.ds(k * CHUNK, CHUNK)
        uk = ids.at[0, ck]
        ik = ids.at[1, ck]
        sem = sem_slot[buf]
        return (pltpu.make_async_copy(p_hbm.at[uk], rows.at[2 * buf], sem),
                pltpu.make_async_copy(q_hbm.at[ik], rows.at[2 * buf + 1], sem),
                pltpu.make_async_copy(ub_hbm.at[uk], small.at[UB, ck], sem),
                pltpu.make_async_copy(ib_hbm.at[ik], small.at[IB, ck], sem))

    def issue(k, buf):
        for cp in chunk_copies(k, buf):
            cp.start()

    def drain(k, buf):
        for cp in chunk_copies(k, buf):
            cp.wait()

    cp_u0 = pltpu.async_copy(uid_hbm.at[pl.ds(base, CHUNK)],
                             ids.at[0, pl.ds(0, CHUNK)], sem_ids)
    cp_i0 = pltpu.async_copy(iid_hbm.at[pl.ds(base, CHUNK)],
                             ids.at[1, pl.ds(0, CHUNK)], sem_ids)
    cp_u0.wait()
    cp_i0.wait()
    issue(0, 0)

    rest = BPW - CHUNK
    cp_ur = pltpu.async_copy(uid_hbm.at[pl.ds(base + CHUNK, rest)],
                             ids.at[0, pl.ds(CHUNK, rest)], sem_ids)
    cp_ir = pltpu.async_copy(iid_hbm.at[pl.ds(base + CHUNK, rest)],
                             ids.at[1, pl.ds(CHUNK, rest)], sem_ids)
    cp_av = pltpu.async_copy(avg_hbm.at[pl.ds(base, BPW)],
                             small.at[AVG], sem_ids)
    cp_ur.wait()
    cp_ir.wait()
    cp_av.wait()
    issue(1, 1)

    lane = lax.iota(jnp.int32, L)
    ub16 = jnp.full((L,), UB, jnp.int32)
    ib16 = jnp.full((L,), IB, jnp.int32)
    av16 = jnp.full((L,), AVG, jnp.int32)
    ot16 = jnp.full((L,), OUT, jnp.int32)

    def compute_chunk(k, buf):
        p_rows, q_rows = rows.at[2 * buf], rows.at[2 * buf + 1]

        @pl.loop(0, GROUPS)
        def _(g):
            rows16 = lane + g * L

            def dbody(m, accs, rows16=rows16):
                a0, a1 = accs
                c0 = m * UNROLL
                for u in range(UNROLL):
                    col = (lane + (c0 + u)) & (LATENT - 1)
                    pv = plsc.load_gather(p_rows, [rows16, col])
                    qv = plsc.load_gather(q_rows, [rows16, col])
                    if u % 2 == 0:
                        a0 = a0 + pv * qv
                    else:
                        a1 = a1 + pv * qv
                return a0, a1

            zero = jnp.zeros((L,), jnp.float32)
            a0, a1 = lax.fori_loop(0, LATENT // UNROLL, dbody, (zero, zero))
            bidx = rows16 + k * CHUNK
            ubv = plsc.load_gather(small, [ub16, bidx])
            ibv = plsc.load_gather(small, [ib16, bidx])
            avv = plsc.load_gather(small, [av16, bidx])
            plsc.store_scatter(small, [ot16, bidx], (a0 + a1) + ubv + ibv + avv)

        pltpu.make_async_copy(
            small.at[OUT, pl.ds(k * CHUNK, CHUNK)],
            out_hbm.at[pl.ds(base + k * CHUNK, CHUNK)], sem_out).start()

    @pl.loop(0, NCHUNK, step=2)
    def _(kb):
        for half in range(2):
            k = kb + half
            drain(k, half)
            compute_chunk(k, half)

            @pl.when(k + 2 < NCHUNK)
            def _():
                issue(k + 2, half)

    for k in range(NCHUNK):
        pltpu.make_async_copy(
            small.at[OUT, pl.ds(k * CHUNK, CHUNK)],
            out_hbm.at[pl.ds(base + k * CHUNK, CHUNK)], sem_out).wait()


def _mf(user_id, item_id, avg, P, Q, ub, ib):
    mesh = plsc.VectorSubcoreMesh(core_axis_name="c", subcore_axis_name="s")
    return pl.kernel(
        _mf_body,
        mesh=mesh,
        compiler_params=pltpu.CompilerParams(needs_layout_passes=False),
        out_type=jax.ShapeDtypeStruct((BATCH,), jnp.float32),
        scratch_types=[
            pltpu.VMEM((2, BPW), jnp.int32),
            pltpu.VMEM((4, CHUNK, LATENT), jnp.float32),
            pltpu.VMEM((4, BPW), jnp.float32),
            pltpu.SemaphoreType.DMA,
            pltpu.SemaphoreType.DMA,
            pltpu.SemaphoreType.DMA,
            pltpu.SemaphoreType.DMA,
        ],
    )(user_id, item_id, avg, P, Q, ub, ib)


def kernel(user_id, item_id, avg_score, P, Q, user_bias, item_bias):
    out = _mf(user_id.astype(jnp.int32), item_id.astype(jnp.int32),
              avg_score.reshape(-1), P, Q,
              user_bias.reshape(-1), item_bias.reshape(-1))
    return out.reshape(BATCH, 1)


# speedup vs baseline: 1.1508x; 1.0406x over previous
import jax
import jax.numpy as jnp
from jax import lax
from jax.experimental import pallas as pl
from jax.experimental.pallas import tpu as pltpu
from jax.experimental.pallas import tpu_sc as plsc

BATCH = 16384
LATENT = 128
NC = 2
NS = 16
L = 16
NW = NC * NS
BPW = BATCH // NW
CHUNK = 128
NCHUNK = BPW // CHUNK
GROUPS = CHUNK // L
UNROLL = 16


def _mf_body(uid_hbm, iid_hbm, avg_hbm, p_hbm, q_hbm, ub_hbm, ib_hbm,
             out_hbm, ids, rows, small, sem0, sem1, sem_ids, sem_out):
    wid = lax.axis_index("s") * NC + lax.axis_index("c")
    base = wid * BPW
    sem_slot = (sem0, sem1)
    UB, IB, AVG, OUT = 0, 1, 2, 3

    def chunk_copies(k, buf):
        ck = pl.ds(k * CHUNK, CHUNK)
        uk = ids.at[0, ck]
        ik = ids.at[1, ck]
        sem = sem_slot[buf]
        return (pltpu.make_async_copy(p_hbm.at[uk], rows.at[2 * buf], sem),
                pltpu.make_async_copy(q_hbm.at[ik], rows.at[2 * buf + 1], sem),
                pltpu.make_async_copy(ub_hbm.at[uk], small.at[UB, ck], sem),
                pltpu.make_async_copy(ib_hbm.at[ik], small.at[IB, ck], sem))

    def issue(k, buf):
        for cp in chunk_copies(k, buf):
            cp.start()

    def drain(k, buf):
        for cp in chunk_copies(k, buf):
            cp.wait()

    cp_u0 = pltpu.async_copy(uid_hbm.at[pl.ds(base, CHUNK)],
                             ids.at[0, pl.ds(0, CHUNK)], sem_ids)
    cp_i0 = pltpu.async_copy(iid_hbm.at[pl.ds(base, CHUNK)],
                             ids.at[1, pl.ds(0, CHUNK)], sem_ids)
    cp_u0.wait()
    cp_i0.wait()
    issue(0, 0)

    rest = BPW - CHUNK
    cp_ur = pltpu.async_copy(uid_hbm.at[pl.ds(base + CHUNK, rest)],
                             ids.at[0, pl.ds(CHUNK, rest)], sem_ids)
    cp_ir = pltpu.async_copy(iid_hbm.at[pl.ds(base + CHUNK, rest)],
                             ids.at[1, pl.ds(CHUNK, rest)], sem_ids)
    cp_av = pltpu.async_copy(avg_hbm.at[pl.ds(base, BPW)],
                             small.at[AVG], sem_ids)
    cp_ur.wait()
    cp_ir.wait()
    cp_av.wait()
    issue(1, 1)

    lane = lax.iota(jnp.int32, L)
    ub16 = jnp.full((L,), UB, jnp.int32)
    ib16 = jnp.full((L,), IB, jnp.int32)
    av16 = jnp.full((L,), AVG, jnp.int32)
    ot16 = jnp.full((L,), OUT, jnp.int32)

    def compute_chunk(k, buf):
        p_rows, q_rows = rows.at[2 * buf], rows.at[2 * buf + 1]

        @pl.loop(0, GROUPS)
        def _(g):
            rows16 = lane + g * L

            def dbody(m, accs, rows16=rows16):
                a0, a1 = accs
                c0 = m * UNROLL
                for u in range(UNROLL):
                    col = (lane + (c0 + u)) & (LATENT - 1)
                    pv = plsc.load_gather(p_rows, [rows16, col])
                    qv = plsc.load_gather(q_rows, [rows16, col])
                    if u % 2 == 0:
                        a0 = a0 + pv * qv
                    else:
                        a1 = a1 + pv * qv
                return a0, a1

            zero = jnp.zeros((L,), jnp.float32)
            a0, a1 = lax.fori_loop(0, LATENT // UNROLL, dbody, (zero, zero))
            bidx = rows16 + k * CHUNK
            ubv = plsc.load_gather(small, [ub16, bidx])
            ibv = plsc.load_gather(small, [ib16, bidx])
            avv = plsc.load_gather(small, [av16, bidx])
            plsc.store_scatter(small, [ot16, bidx], (a0 + a1) + ubv + ibv + avv)

        pltpu.make_async_copy(
            small.at[OUT, pl.ds(k * CHUNK, CHUNK)],
            out_hbm.at[pl.ds(base + k * CHUNK, CHUNK)], sem_out).start()

    @pl.loop(0, NCHUNK, step=2)
    def _(kb):
        for half in range(2):
            k = kb + half
            drain(k, half)
            compute_chunk(k, half)

            @pl.when(k + 2 < NCHUNK)
            def _():
                issue(k + 2, half)

    for k in range(NCHUNK):
        pltpu.make_async_copy(
            small.at[OUT, pl.ds(k * CHUNK, CHUNK)],
            out_hbm.at[pl.ds(base + k * CHUNK, CHUNK)], sem_out).wait()


def _mf(user_id, item_id, avg, P, Q, ub, ib):
    mesh = plsc.VectorSubcoreMesh(core_axis_name="c", subcore_axis_name="s")
    return pl.kernel(
        _mf_body,
        mesh=mesh,
        compiler_params=pltpu.CompilerParams(needs_layout_passes=False),
        out_type=jax.ShapeDtypeStruct((BATCH,), jnp.float32),
        scratch_types=[
            pltpu.VMEM((2, BPW), jnp.int32),
            pltpu.VMEM((4, CHUNK, LATENT), jnp.float32),
            pltpu.VMEM((4, BPW), jnp.float32),
            pltpu.SemaphoreType.DMA,
            pltpu.SemaphoreType.DMA,
            pltpu.SemaphoreType.DMA,
            pltpu.SemaphoreType.DMA,
        ],
    )(user_id, item_id, avg, P, Q, ub, ib)


def kernel(user_id, item_id, avg_score, P, Q, user_bias, item_bias):
    pad = (-user_bias.shape[0]) % 1024
    ubp = jnp.pad(user_bias, ((0, pad), (0, 0))).reshape(-1)
    ibp = jnp.pad(item_bias, ((0, pad), (0, 0))).reshape(-1)
    out = _mf(user_id.astype(jnp.int32), item_id.astype(jnp.int32),
              avg_score.reshape(-1), P, Q, ubp, ibp)
    return out.reshape(BATCH, 1)
